# baseline (speedup 1.0000x reference)
import functools

import jax
import jax.numpy as jnp
from jax import lax
from jax.experimental import pallas as pl
from jax.experimental.pallas import tpu as pltpu
from jax.experimental.pallas import tpu_sc as plsc

HIDDEN = 2048
NUM_EXPERTS = 64
TOPK = 8
TOPK_SCALING_FACTOR = 2.5
BLK = 2048

N_TOKENS = 16384
N_WORKERS = 32
ROWS_PER_W = N_TOKENS // N_WORKERS
LANES = 16
GROUPS = ROWS_PER_W // LANES


def _scores_body(x_ref, w_ref, s_ref):
    x = x_ref[...]
    logits = lax.dot_general(
        x, w_ref[...], (((1,), (1,)), ((), ())),
        preferred_element_type=jnp.float32,
    )
    sp = jnp.maximum(logits, 0.0) + jnp.log(1.0 + jnp.exp(-jnp.abs(logits)))
    s_ref[...] = jnp.sqrt(sp)


def _tc_scores(flat, weight):
    n = flat.shape[0]
    return pl.pallas_call(
        _scores_body,
        grid=(n // BLK,),
        in_specs=[
            pl.BlockSpec((BLK, HIDDEN), lambda i: (i, 0)),
            pl.BlockSpec((NUM_EXPERTS, HIDDEN), lambda i: (0, 0)),
        ],
        out_specs=pl.BlockSpec((BLK, NUM_EXPERTS), lambda i: (i, 0)),
        out_shape=jax.ShapeDtypeStruct((n, NUM_EXPERTS), jnp.float32),
    )(flat, weight)


def _route_body(scores_hbm, bias_hbm, probs_hbm, route_hbm,
                sbuf, wbuf, pbuf, rbuf, bbuf):
    wid = lax.axis_index("s") * 2 + lax.axis_index("c")
    base = wid * (ROWS_PER_W * NUM_EXPERTS)
    rbase = wid * (ROWS_PER_W * 16)

    pltpu.sync_copy(scores_hbm.at[pl.ds(base, ROWS_PER_W * NUM_EXPERTS)], sbuf)
    pltpu.sync_copy(bias_hbm, bbuf)

    lanes = jnp.arange(LANES, dtype=jnp.int32)
    zf = jnp.zeros((LANES,), jnp.float32)
    zi = jnp.zeros((LANES,), jnp.int32)

    def _bias_row(i, carry):
        for j in range(NUM_EXPERTS // LANES):
            off = i * NUM_EXPERTS + j * LANES
            bj = bbuf[pl.ds(j * LANES, LANES)]
            wbuf[pl.ds(off, LANES)] = sbuf[pl.ds(off, LANES)] + bj
        return carry
    lax.fori_loop(0, ROWS_PER_W, _bias_row, 0)

    def _zero_p(i, carry):
        pbuf[pl.ds(i * LANES, LANES)] = zf
        return carry
    lax.fori_loop(0, ROWS_PER_W * NUM_EXPERTS // LANES, _zero_p, 0)

    def _zero_r(i, carry):
        rbuf[pl.ds(i * LANES, LANES)] = zi
        return carry
    lax.fori_loop(0, ROWS_PER_W * 16 // LANES, _zero_r, 0)

    neg_inf = jnp.full((LANES,), -jnp.inf, jnp.float32)

    def _group(g, carry):
        off0 = g * (LANES * NUM_EXPERTS)
        row_off = lanes * NUM_EXPERTS + off0
        picked_idx = []
        picked_w = []
        denom = zf
        for _ in range(TOPK):
            best = neg_inf
            bidx = zi
            for e in range(NUM_EXPERTS):
                v = plsc.load_gather(wbuf, [row_off + e])
                c = v > best
                best = jnp.where(c, v, best)
                bidx = jnp.where(c, e, bidx)
            idx_ch = row_off + bidx
            plsc.store_scatter(wbuf, [idx_ch], neg_inf)
            w = plsc.load_gather(sbuf, [idx_ch])
            denom = denom + w
            picked_idx.append((idx_ch, bidx))
            picked_w.append(w)
        scale = TOPK_SCALING_FACTOR / jnp.maximum(denom, 1e-12)
        rrow = g * (LANES * 16) + lanes * 16
        one = jnp.full((LANES,), 1, jnp.int32)
        for (idx_ch, bidx), w in zip(picked_idx, picked_w):
            plsc.store_scatter(pbuf, [idx_ch], w * scale)
            byte = one << ((bidx & 3) << 3)
            plsc.addupdate_scatter(rbuf, [rrow + (bidx >> 2)], byte)
        return carry
    lax.fori_loop(0, GROUPS, _group, 0)

    pltpu.sync_copy(pbuf, probs_hbm.at[pl.ds(base, ROWS_PER_W * NUM_EXPERTS)])
    pltpu.sync_copy(rbuf, route_hbm.at[pl.ds(rbase, ROWS_PER_W * 16)])


@functools.partial(jax.jit, static_argnums=())
def _sc_route(scores_flat, expert_bias):
    mesh = plsc.VectorSubcoreMesh(core_axis_name="c", subcore_axis_name="s")
    run = pl.kernel(
        _route_body,
        mesh=mesh,
        compiler_params=pltpu.CompilerParams(needs_layout_passes=False),
        out_type=[
            jax.ShapeDtypeStruct((N_TOKENS * NUM_EXPERTS,), jnp.float32),
            jax.ShapeDtypeStruct((N_TOKENS * 16,), jnp.int32),
        ],
        scratch_types=[
            pltpu.VMEM((ROWS_PER_W * NUM_EXPERTS,), jnp.float32),
            pltpu.VMEM((ROWS_PER_W * NUM_EXPERTS,), jnp.float32),
            pltpu.VMEM((ROWS_PER_W * NUM_EXPERTS,), jnp.float32),
            pltpu.VMEM((ROWS_PER_W * 16,), jnp.int32),
            pltpu.VMEM((NUM_EXPERTS,), jnp.float32),
        ],
    )
    return run(scores_flat, expert_bias)


def kernel(hidden, weight, expert_bias):
    flat = hidden.reshape(-1, HIDDEN)
    scores = _tc_scores(flat, weight)
    probs_flat, route_words = _sc_route(scores.reshape(-1), expert_bias)
    probs = probs_flat.reshape(N_TOKENS, NUM_EXPERTS)
    route_bytes = lax.bitcast_convert_type(route_words, jnp.int8)
    rmap = route_bytes.reshape(N_TOKENS, NUM_EXPERTS) != 0
    return probs, rmap


# speedup vs baseline: 4.0349x; 4.0349x over previous
import jax
import jax.numpy as jnp
from jax.experimental import pallas as pl

HIDDEN = 2048
NUM_EXPERTS = 64
TOPK = 8
TOPK_SCALING_FACTOR = 2.5
BLK = 2048


def _router_body(x_ref, wt_ref, b_ref, probs_ref, map_ref):
    x = x_ref[...]
    logits = jax.lax.dot_general(
        x, wt_ref[...], (((1,), (1,)), ((), ())),
        preferred_element_type=jnp.float32,
    )
    sp = jnp.maximum(logits, 0.0) + jnp.log(1.0 + jnp.exp(-jnp.abs(logits)))
    scores = jnp.sqrt(sp)
    sel = scores + b_ref[...]
    iota = jax.lax.broadcasted_iota(jnp.int32, sel.shape, 1)
    mask = jnp.zeros(sel.shape, jnp.bool_)
    work = sel
    for _ in range(TOPK):
        idx = jnp.argmax(work, axis=1)[:, None]
        chosen = iota == idx
        mask = jnp.logical_or(mask, chosen)
        work = jnp.where(chosen, -jnp.inf, work)
    w = jnp.where(mask, scores, 0.0)
    denom = jnp.clip(jnp.sum(w, axis=1, keepdims=True), 1e-12, None)
    probs_ref[...] = jnp.where(mask, scores * (TOPK_SCALING_FACTOR / denom), 0.0)
    map_ref[...] = mask.astype(jnp.int8)


def kernel(hidden, weight, expert_bias):
    flat = hidden.reshape(-1, HIDDEN)
    n = flat.shape[0]
    bias = expert_bias.reshape(1, NUM_EXPERTS)
    probs, rmap = pl.pallas_call(
        _router_body,
        grid=(n // BLK,),
        in_specs=[
            pl.BlockSpec((BLK, HIDDEN), lambda i: (i, 0)),
            pl.BlockSpec((NUM_EXPERTS, HIDDEN), lambda i: (0, 0)),
            pl.BlockSpec((1, NUM_EXPERTS), lambda i: (0, 0)),
        ],
        out_specs=[
            pl.BlockSpec((BLK, NUM_EXPERTS), lambda i: (i, 0)),
            pl.BlockSpec((BLK, NUM_EXPERTS), lambda i: (i, 0)),
        ],
        out_shape=[
            jax.ShapeDtypeStruct((n, NUM_EXPERTS), jnp.float32),
            jax.ShapeDtypeStruct((n, NUM_EXPERTS), jnp.int8),
        ],
    )(flat, weight, bias)
    return probs, rmap.astype(jnp.bool_)
